# Initial kernel scaffold; baseline (speedup 1.0000x reference)
import jax
import jax.numpy as jnp
from jax.experimental import pallas as pl


def kernel(input):
    raise NotImplementedError("write your pallas kernel here")


# speedup vs baseline: 19.0747x; 19.0747x over previous
import functools

import jax
import jax.numpy as jnp
from jax import lax
from jax.experimental import pallas as pl
from jax.experimental.pallas import tpu as pltpu
from jax.experimental.pallas import tpu_sc as plsc

NC, NS, L = 2, 16, 16
NW = NC * NS
T, B, N = 8, 128, 4096
R = T * B
RPW = R // NW
GROUP = 8
CHUNKS = N // L
UNROLL = 8


def _dyn_gather(v, idx):
    return lax.gather(
        v,
        idx[:, None],
        dimension_numbers=lax.GatherDimensionNumbers(
            offset_dims=(), collapsed_slice_dims=(0,), start_index_map=(0,)
        ),
        slice_sizes=(1,),
        mode=lax.GatherScatterMode.PROMISE_IN_BOUNDS,
    )


def _tree_sum(acc, iota):
    for st in (8, 4, 2, 1):
        perm = lax.rem(iota + st, L)
        acc = acc + _dyn_gather(acc, perm)
    return acc


def _sc_body(x_hbm, out_hbm, buf, outv):
    wid = lax.axis_index("s") * NC + lax.axis_index("c")
    base_row = wid * RPW
    iota = lax.iota(jnp.int32, L)
    one = jnp.full((L,), 1, jnp.int32)
    zero = jnp.full((L,), 0, jnp.int32)

    c1 = zero
    c5 = zero
    c10 = zero
    for g in range(RPW // GROUP):
        pltpu.sync_copy(x_hbm.at[pl.ds((base_row + g * GROUP) * N, GROUP * N)], buf)

        def row_body(rl, carry, g=g):
            c1, c5, c10 = carry
            r = base_row + g * GROUP + rl
            i = lax.rem(r, B)
            off = rl * N

            lane = lax.rem(i, L)
            chunk_d = buf[pl.ds(off + (i - lane), L)]
            d = _dyn_gather(chunk_d, jnp.full((L,), lane, jnp.int32))

            ilim = jnp.full((L,), i, jnp.int32)
            acc = zero
            for j in range(B // L):
                v = buf[pl.ds(off + j * L, L)]
                col = iota + (j * L)
                acc = acc + jnp.where((v == d) & (col < ilim), one, zero)

            def chunk(jc, acc):
                base = off + jc * (UNROLL * L)
                for q in range(UNROLL):
                    v = buf[pl.ds(base + q * L, L)]
                    acc = acc + jnp.where(v > d, one, zero)
                return acc

            acc = lax.fori_loop(0, CHUNKS // UNROLL, chunk, acc)
            rank = _tree_sum(acc, iota)

            c1 = c1 + jnp.where(rank < 1, one, zero)
            c5 = c5 + jnp.where(rank < 5, one, zero)
            c10 = c10 + jnp.where(rank < 10, one, zero)
            return (c1, c5, c10)

        c1, c5, c10 = lax.fori_loop(0, GROUP, row_body, (c1, c5, c10))

    res = (
        jnp.where(iota == 0, c1, zero)
        + jnp.where(iota == 1, c5, zero)
        + jnp.where(iota == 2, c10, zero)
    ).astype(jnp.float32)
    outv[...] = res
    pltpu.sync_copy(outv, out_hbm.at[wid])


@jax.jit
def kernel(input):
    x = input.reshape(R * N)
    mesh = plsc.VectorSubcoreMesh(
        core_axis_name="c", subcore_axis_name="s", num_cores=NC, num_subcores=NS
    )
    partial = pl.kernel(
        _sc_body,
        out_type=jax.ShapeDtypeStruct((NW, L), jnp.float32),
        mesh=mesh,
        scratch_types=[
            pltpu.VMEM((GROUP * N,), jnp.float32),
            pltpu.VMEM((L,), jnp.float32),
        ],
    )(x)
    sums = jnp.sum(partial[:, :3], axis=0)
    return (sums[0] / R, sums[1] / R, sums[2] / R)


# speedup vs baseline: 26.9101x; 1.4108x over previous
import functools

import jax
import jax.numpy as jnp
from jax import lax
from jax.experimental import pallas as pl
from jax.experimental.pallas import tpu as pltpu
from jax.experimental.pallas import tpu_sc as plsc

NC, NS, L = 2, 16, 16
NW = NC * NS
T, B, N = 8, 128, 4096
R = T * B
RPW = R // NW
GROUP = 8
CHUNKS = N // L
UNROLL = 8


def _dyn_gather(v, idx):
    return lax.gather(
        v,
        idx[:, None],
        dimension_numbers=lax.GatherDimensionNumbers(
            offset_dims=(), collapsed_slice_dims=(0,), start_index_map=(0,)
        ),
        slice_sizes=(1,),
        mode=lax.GatherScatterMode.PROMISE_IN_BOUNDS,
    )


def _tree_sum(acc, iota):
    for st in (8, 4, 2, 1):
        perm = lax.rem(iota + st, L)
        acc = acc + _dyn_gather(acc, perm)
    return acc


def _sc_body(x_hbm, out_hbm, buf, outv):
    wid = lax.axis_index("s") * NC + lax.axis_index("c")
    t = wid // (B // RPW)
    r_base = lax.rem(wid, B // RPW) * RPW
    iota = lax.iota(jnp.int32, L)
    one = jnp.full((L,), 1, jnp.int32)
    zero = jnp.full((L,), 0, jnp.int32)

    c1 = zero
    c5 = zero
    c10 = zero
    for g in range(RPW // GROUP):
        pltpu.sync_copy(x_hbm.at[t, pl.ds(r_base + g * GROUP, GROUP), :], buf)

        def row_body(rl, carry, g=g):
            c1, c5, c10 = carry
            i = r_base + g * GROUP + rl

            lane = lax.rem(i, L)
            chunk_d = buf[rl, pl.ds(pl.multiple_of(i - lane, L), L)]
            d = _dyn_gather(chunk_d, jnp.full((L,), lane, jnp.int32))

            ilim = jnp.full((L,), i, jnp.int32)
            acc = zero
            for j in range(B // L):
                v = buf[rl, pl.ds(j * L, L)]
                col = iota + (j * L)
                acc = acc + jnp.where((v == d) & (col < ilim), one, zero)

            def chunk(jc, acc):
                base = pl.multiple_of(jc * (UNROLL * L), UNROLL * L)
                for q in range(UNROLL):
                    v = buf[rl, pl.ds(pl.multiple_of(base + q * L, L), L)]
                    acc = acc + jnp.where(v > d, one, zero)
                return acc

            acc = lax.fori_loop(0, CHUNKS // UNROLL, chunk, acc)
            rank = _tree_sum(acc, iota)

            c1 = c1 + jnp.where(rank < 1, one, zero)
            c5 = c5 + jnp.where(rank < 5, one, zero)
            c10 = c10 + jnp.where(rank < 10, one, zero)
            return (c1, c5, c10)

        c1, c5, c10 = lax.fori_loop(0, GROUP, row_body, (c1, c5, c10))

    res = (
        jnp.where(iota == 0, c1, zero)
        + jnp.where(iota == 1, c5, zero)
        + jnp.where(iota == 2, c10, zero)
    ).astype(jnp.float32)
    outv[...] = res
    pltpu.sync_copy(outv, out_hbm.at[wid])


@jax.jit
def kernel(input):
    mesh = plsc.VectorSubcoreMesh(
        core_axis_name="c", subcore_axis_name="s", num_cores=NC, num_subcores=NS
    )
    partial = pl.kernel(
        _sc_body,
        out_type=jax.ShapeDtypeStruct((NW, L), jnp.float32),
        mesh=mesh,
        scratch_types=[
            pltpu.VMEM((GROUP, N), jnp.float32),
            pltpu.VMEM((L,), jnp.float32),
        ],
        compiler_params=pltpu.CompilerParams(use_tc_tiling_on_sc=True),
    )(input)
    sums = jnp.sum(partial[:, :3], axis=0)
    return (sums[0] / R, sums[1] / R, sums[2] / R)


# speedup vs baseline: 29.9261x; 1.1121x over previous
import functools

import jax
import jax.numpy as jnp
from jax import lax
from jax.experimental import pallas as pl
from jax.experimental.pallas import tpu as pltpu
from jax.experimental.pallas import tpu_sc as plsc

NC, NS, L = 2, 16, 16
NW = NC * NS
T, B, N = 8, 128, 4096
R = T * B
RPW = R // NW
GROUP = 8
CHUNKS = N // L
UNROLL = 8


def _dyn_gather(v, idx):
    return lax.gather(
        v,
        idx[:, None],
        dimension_numbers=lax.GatherDimensionNumbers(
            offset_dims=(), collapsed_slice_dims=(0,), start_index_map=(0,)
        ),
        slice_sizes=(1,),
        mode=lax.GatherScatterMode.PROMISE_IN_BOUNDS,
    )


def _tree_sum(acc, iota):
    for st in (8, 4, 2, 1):
        perm = lax.rem(iota + st, L)
        acc = acc + _dyn_gather(acc, perm)
    return acc


def _sc_body(x_hbm, out_hbm, buf_a, buf_b, outv, sem_a, sem_b):
    wid = lax.axis_index("s") * NC + lax.axis_index("c")
    t = wid // (B // RPW)
    r_base = lax.rem(wid, B // RPW) * RPW
    iota = lax.iota(jnp.int32, L)
    one = jnp.full((L,), 1, jnp.int32)
    zero = jnp.full((L,), 0, jnp.int32)

    def copy(g, buf, sem):
        return pltpu.async_copy(
            x_hbm.at[t, pl.ds(r_base + g * GROUP, GROUP), :], buf, sem
        )

    def process(g, buf, carry):
        def row_body(rl, carry):
            c1, c5, c10 = carry
            i = r_base + g * GROUP + rl

            lane = lax.rem(i, L)
            chunk_d = buf[rl, pl.ds(pl.multiple_of(i - lane, L), L)]
            d = _dyn_gather(chunk_d, jnp.full((L,), lane, jnp.int32))

            ilim = jnp.full((L,), i, jnp.int32)
            acc = zero
            for j in range(B // L):
                v = buf[rl, pl.ds(j * L, L)]
                col = iota + (j * L)
                acc = acc + jnp.where((v == d) & (col < ilim), one, zero)

            def chunk(jc, acc):
                base = pl.multiple_of(jc * (UNROLL * L), UNROLL * L)
                for q in range(UNROLL):
                    v = buf[rl, pl.ds(pl.multiple_of(base + q * L, L), L)]
                    acc = acc + jnp.where(v > d, one, zero)
                return acc

            acc = lax.fori_loop(0, CHUNKS // UNROLL, chunk, acc)
            rank = _tree_sum(acc, iota)

            c1 = c1 + jnp.where(rank < 1, one, zero)
            c5 = c5 + jnp.where(rank < 5, one, zero)
            c10 = c10 + jnp.where(rank < 10, one, zero)
            return (c1, c5, c10)

        return lax.fori_loop(0, GROUP, row_body, carry)

    carry = (zero, zero, zero)
    h_a = copy(0, buf_a, sem_a)
    h_a.wait()
    h_b = copy(1, buf_b, sem_b)
    carry = process(0, buf_a, carry)
    h_b.wait()
    h_a = copy(2, buf_a, sem_a)
    carry = process(1, buf_b, carry)
    h_a.wait()
    h_b = copy(3, buf_b, sem_b)
    carry = process(2, buf_a, carry)
    h_b.wait()
    carry = process(3, buf_b, carry)
    c1, c5, c10 = carry

    res = (
        jnp.where(iota == 0, c1, zero)
        + jnp.where(iota == 1, c5, zero)
        + jnp.where(iota == 2, c10, zero)
    ).astype(jnp.float32)
    outv[...] = res
    pltpu.sync_copy(outv, out_hbm.at[wid])


@jax.jit
def kernel(input):
    mesh = plsc.VectorSubcoreMesh(
        core_axis_name="c", subcore_axis_name="s", num_cores=NC, num_subcores=NS
    )
    partial = pl.kernel(
        _sc_body,
        out_type=jax.ShapeDtypeStruct((NW, L), jnp.float32),
        mesh=mesh,
        scratch_types=[
            pltpu.VMEM((GROUP, N), jnp.float32),
            pltpu.VMEM((GROUP, N), jnp.float32),
            pltpu.VMEM((L,), jnp.float32),
            pltpu.SemaphoreType.DMA,
            pltpu.SemaphoreType.DMA,
        ],
        compiler_params=pltpu.CompilerParams(use_tc_tiling_on_sc=True),
    )(input)
    sums = jnp.sum(partial[:, :3], axis=0)
    return (sums[0] / R, sums[1] / R, sums[2] / R)
